# Initial kernel scaffold; baseline (speedup 1.0000x reference)
import jax
import jax.numpy as jnp
from jax.experimental import pallas as pl


def kernel(x, edge_idx, W1, b1, W2, b2):
    raise NotImplementedError("write your pallas kernel here")


# speedup vs baseline: 12.2963x; 12.2963x over previous
import functools

import jax
import jax.numpy as jnp
from jax import lax
from jax.experimental import pallas as pl
from jax.experimental.pallas import tpu as pltpu
from jax.experimental.pallas import tpu_sc as plsc

N = 10000
NP = 10240
NFEAT = 128
NHID = 256
NCLASS = 64
K = 128
NC = 2
NS = 16
NW = NC * NS


def _make_deg_kernel(nb):
    per_tile = nb // NW
    rows_per_tile = NP // NS
    mesh = plsc.VectorSubcoreMesh(core_axis_name="c", subcore_axis_name="s",
                                  num_cores=NC, num_subcores=NS)

    @functools.partial(
        pl.kernel,
        out_type=jax.ShapeDtypeStruct((NC, NP), jnp.float32),
        mesh=mesh,
        scratch_types=[
            pltpu.VMEM((K,), jnp.int32),
            pltpu.VMEM((K,), jnp.float32),
            pltpu.VMEM((rows_per_tile,), jnp.float32),
            pltpu.VMEM_SHARED((NP,), jnp.float32),
        ],
    )
    def k(col_hbm, out_hbm, colb, ones, zbuf, acc):
        cid = lax.axis_index("c")
        sid = lax.axis_index("s")
        base = (cid * NS + sid) * per_tile
        row0 = sid * rows_per_tile

        def fill(i, _):
            ones[pl.ds(i * 16, 16)] = jnp.ones((16,), jnp.float32)
            return 0
        lax.fori_loop(0, K // 16, fill, 0)

        def zfill(i, _):
            zbuf[pl.ds(i * 16, 16)] = jnp.zeros((16,), jnp.float32)
            return 0
        lax.fori_loop(0, rows_per_tile // 16, zfill, 0)
        pltpu.sync_copy(zbuf, acc.at[pl.ds(row0, rows_per_tile)])
        plsc.subcore_barrier()

        def body(j, _):
            pltpu.sync_copy(col_hbm.at[base + j], colb)
            pltpu.sync_copy(ones, acc.at[colb], add=True)
            return 0
        lax.fori_loop(0, per_tile, body, 0)
        plsc.subcore_barrier()
        pltpu.sync_copy(acc.at[pl.ds(row0, rows_per_tile)],
                        out_hbm.at[cid, pl.ds(row0, rows_per_tile)])

    return k


def _make_agg_kernel(nb, d):
    per_tile = nb // NW
    rows_per_tile = NP // NS
    mesh = plsc.VectorSubcoreMesh(core_axis_name="c", subcore_axis_name="s",
                                  num_cores=NC, num_subcores=NS)

    @functools.partial(
        pl.kernel,
        out_type=jax.ShapeDtypeStruct((NC, NP, d), jnp.float32),
        mesh=mesh,
        scratch_types=[
            pltpu.VMEM((K,), jnp.int32),
            pltpu.VMEM((K,), jnp.int32),
            pltpu.VMEM((K, d), jnp.float32),
            pltpu.VMEM_SHARED((NP, d), jnp.float32),
            pltpu.SemaphoreType.DMA,
        ],
    )
    def k(row_hbm, col_hbm, xs_hbm, out_hbm, rowb, colb, msg, acc, sem):
        cid = lax.axis_index("c")
        sid = lax.axis_index("s")
        base = (cid * NS + sid) * per_tile
        row0 = sid * rows_per_tile

        def zfill(i, _):
            for j in range(d // 16):
                msg[i, pl.ds(j * 16, 16)] = jnp.zeros((16,), jnp.float32)
            return 0
        lax.fori_loop(0, K, zfill, 0)
        for t in range(rows_per_tile // K):
            pltpu.sync_copy(msg, acc.at[pl.ds(row0 + t * K, K)])
        plsc.subcore_barrier()

        def body(j, _):
            pltpu.sync_copy(row_hbm.at[base + j], rowb)
            pltpu.sync_copy(col_hbm.at[base + j], colb)
            pltpu.async_copy(xs_hbm.at[rowb], msg, sem).wait()
            pltpu.sync_copy(msg, acc.at[colb], add=True)
            return 0
        lax.fori_loop(0, per_tile, body, 0)
        plsc.subcore_barrier()
        pltpu.sync_copy(acc.at[pl.ds(row0, rows_per_tile)],
                        out_hbm.at[cid, pl.ds(row0, rows_per_tile)])

    return k


BR = 400


def _scale_body(deg_ref, x_ref, d_ref, xs_ref):
    deg = deg_ref[:, 0:1] + deg_ref[:, 1:2] + 1.0
    dv = lax.rsqrt(deg)
    d_ref[...] = dv
    xs_ref[...] = x_ref[...] * dv


def _fused_body(p_ref, xs_ref, d_ref, w1_ref, b1_ref, w2_ref, xs2_ref):
    dv = d_ref[...]
    agg = (p_ref[0] + p_ref[1] + xs_ref[...]) * dv
    h = jnp.dot(agg, w1_ref[...], preferred_element_type=jnp.float32)
    h = jnp.maximum(h + b1_ref[...], 0.0) * dv
    t2 = jnp.dot(h, w2_ref[...], preferred_element_type=jnp.float32)
    xs2_ref[...] = jnp.concatenate(
        [t2, jnp.zeros_like(t2)], axis=1)


def _logsmax_body(q_ref, xs2_ref, d_ref, b2_ref, out_ref):
    z = ((q_ref[0, :, :NCLASS] + q_ref[1, :, :NCLASS] + xs2_ref[:, :NCLASS])
         * d_ref[...] + b2_ref[...])
    m = jnp.max(z, axis=1, keepdims=True)
    e = jnp.exp(z - m)
    out_ref[...] = z - m - jnp.log(jnp.sum(e, axis=1, keepdims=True))


@jax.jit
def kernel(x, edge_idx, W1, b1, W2, b2):
    row = edge_idx[0]
    col = edge_idx[1]
    e = row.shape[0]
    nb = (e + NW * K - 1) // (NW * K)
    epad = nb * NW * K - e
    row_p = jnp.concatenate([row, jnp.zeros((epad,), row.dtype)])
    col_p = jnp.concatenate([col, jnp.full((epad,), N, col.dtype)])
    row_b = row_p.reshape(nb * NW, K)
    col_b = col_p.reshape(nb * NW, K)

    degp = _make_deg_kernel(nb * NW)(col_b)

    grid = N // BR
    d, xs = pl.pallas_call(
        _scale_body,
        grid=(grid,),
        in_specs=[
            pl.BlockSpec((BR, NC), lambda r: (r, 0)),
            pl.BlockSpec((BR, NFEAT), lambda r: (r, 0)),
        ],
        out_specs=[
            pl.BlockSpec((BR, 1), lambda r: (r, 0)),
            pl.BlockSpec((BR, NFEAT), lambda r: (r, 0)),
        ],
        out_shape=[
            jax.ShapeDtypeStruct((N, 1), jnp.float32),
            jax.ShapeDtypeStruct((N, NFEAT), jnp.float32),
        ],
    )(degp[:, :N].T, x)

    p = _make_agg_kernel(nb * NW, NFEAT)(row_b, col_b, xs)

    xs2 = pl.pallas_call(
        _fused_body,
        grid=(grid,),
        in_specs=[
            pl.BlockSpec((NC, BR, NFEAT), lambda r: (0, r, 0)),
            pl.BlockSpec((BR, NFEAT), lambda r: (r, 0)),
            pl.BlockSpec((BR, 1), lambda r: (r, 0)),
            pl.BlockSpec((NFEAT, NHID), lambda r: (0, 0)),
            pl.BlockSpec((1, NHID), lambda r: (0, 0)),
            pl.BlockSpec((NHID, NCLASS), lambda r: (0, 0)),
        ],
        out_specs=pl.BlockSpec((BR, 2 * NCLASS), lambda r: (r, 0)),
        out_shape=jax.ShapeDtypeStruct((N, 2 * NCLASS), jnp.float32),
    )(p, xs, d, W1, b1.reshape(1, NHID), W2)

    q = _make_agg_kernel(nb * NW, 2 * NCLASS)(row_b, col_b, xs2)

    out = pl.pallas_call(
        _logsmax_body,
        grid=(grid,),
        in_specs=[
            pl.BlockSpec((NC, BR, 2 * NCLASS), lambda r: (0, r, 0)),
            pl.BlockSpec((BR, 2 * NCLASS), lambda r: (r, 0)),
            pl.BlockSpec((BR, 1), lambda r: (r, 0)),
            pl.BlockSpec((1, NCLASS), lambda r: (0, 0)),
        ],
        out_specs=pl.BlockSpec((BR, NCLASS), lambda r: (r, 0)),
        out_shape=jax.ShapeDtypeStruct((N, NCLASS), jnp.float32),
    )(q, xs2, d, b2.reshape(1, NCLASS))

    return out
